# Initial kernel scaffold; baseline (speedup 1.0000x reference)
import jax
import jax.numpy as jnp
from jax.experimental import pallas as pl


def kernel(x, table):
    raise NotImplementedError("write your pallas kernel here")


# speedup vs baseline: 1.2277x; 1.2277x over previous
import jax
import jax.numpy as jnp
from jax import lax
from jax.experimental import pallas as pl
from jax.experimental.pallas import tpu as pltpu
from jax.experimental.pallas import tpu_sc as plsc

B = 16384
L = 200
H = 100

NC = 2
NS = 16
NW = NC * NS
ROWS_W = B // NW
CH = 128
NCHUNK = ROWS_W // CH
CW = CH * L

_ROW_OFFS = (0, 16, 32, 48, 64, 80, 84)


def _body(x_hbm, table_hbm, out_hbm, x_v, out_v, sem):
    wid = lax.axis_index("s") * NC + lax.axis_index("c")

    def chunk(k, carry):
        base = (wid * ROWS_W + k * CH) * L
        pltpu.sync_copy(x_hbm.at[pl.ds(base, CW)], x_v)

        def fire(r, c):
            pltpu.async_copy(
                table_hbm.at[x_v.at[pl.ds(r * L, H)]],
                out_v.at[pl.ds(r * L, H)],
                sem,
            )
            return c

        lax.fori_loop(0, CH, fire, 0)

        def cast_row(r, c):
            for o in _ROW_OFFS:
                s = pl.ds(r * L + H + o, 16)
                out_v[s] = x_v[s].astype(jnp.float32)
            return c

        lax.fori_loop(0, CH, cast_row, 0)

        pltpu.make_async_copy(
            x_hbm.at[pl.ds(0, CH * H)], x_v.at[pl.ds(0, CH * H)], sem
        ).wait()
        pltpu.sync_copy(out_v, out_hbm.at[pl.ds(base, CW)])
        return carry

    lax.fori_loop(0, NCHUNK, chunk, 0)


def kernel(x, table):
    x_flat = x.reshape(-1)
    table1 = table.reshape(-1)
    mesh = plsc.VectorSubcoreMesh(core_axis_name="c", subcore_axis_name="s")
    run = pl.kernel(
        _body,
        out_type=jax.ShapeDtypeStruct((B * L,), jnp.float32),
        mesh=mesh,
        scratch_types=[
            pltpu.VMEM((CW,), jnp.int32),
            pltpu.VMEM((CW,), jnp.float32),
            pltpu.SemaphoreType.DMA,
        ],
    )
    return run(x_flat, table1).reshape(B, L)
